# Initial kernel scaffold; baseline (speedup 1.0000x reference)
import jax
import jax.numpy as jnp
from jax.experimental import pallas as pl


def kernel(x, batch, W1, b1, W2, b2, Wl1, bl1, Wl2, bl2):
    raise NotImplementedError("write your pallas kernel here")


# speedup vs baseline: 7.3544x; 7.3544x over previous
import jax
import jax.numpy as jnp
from jax.experimental import pallas as pl
from jax.experimental.pallas import tpu as pltpu

N = 100000
D = 128
G = 512
BLK = 2000
NBLK = N // BLK


def _body(x_ref, ids_ref, w1_ref, b1_ref, wc_ref, bc_ref, bc0_ref, out_ref,
          acc_ref):
    i = pl.program_id(0)

    @pl.when(i == 0)
    def _init():
        acc_ref[...] = jnp.zeros_like(acc_ref)

    h = jnp.dot(x_ref[...], w1_ref[...], preferred_element_type=jnp.float32)
    h = jnp.maximum(h + b1_ref[...], 0.0)
    y = jnp.dot(h, wc_ref[...], preferred_element_type=jnp.float32)
    y3 = jnp.concatenate([y, jnp.ones((BLK, 1), jnp.float32)], axis=1)
    ids = ids_ref[0, :, :]
    onehot = (jax.lax.broadcasted_iota(jnp.int32, (G, BLK), 0)
              == ids).astype(jnp.float32)
    acc_ref[...] += jnp.dot(onehot, y3, preferred_element_type=jnp.float32)

    @pl.when(i == NBLK - 1)
    def _fin():
        acc = acc_ref[...]
        counts = acc[:, 2:3]
        z = acc[:, 0:2] / jnp.maximum(counts, 1.0) + bc_ref[...]
        z = jnp.where(counts > 0.0, z, bc0_ref[...])
        zmax = jnp.max(z, axis=0, keepdims=True)
        e = jnp.exp(z - zmax)
        out_ref[...] = e / jnp.sum(e, axis=0, keepdims=True)


def kernel(x, batch, W1, b1, W2, b2, Wl1, bl1, Wl2, bl2):
    ids3 = batch.astype(jnp.int32).reshape(NBLK, 1, BLK)
    Wm = Wl1 @ Wl2
    Wc = W2 @ Wm
    bc0 = bl1 @ Wl2 + bl2
    bc = (b2 @ Wm + bc0).reshape(1, 2)
    bc0 = bc0.reshape(1, 2)
    b1_2d = b1.reshape(1, 64)
    return pl.pallas_call(
        _body,
        grid=(NBLK,),
        in_specs=[
            pl.BlockSpec((BLK, D), lambda i: (i, 0)),
            pl.BlockSpec((1, 1, BLK), lambda i: (i, 0, 0)),
            pl.BlockSpec((D, 64), lambda i: (0, 0)),
            pl.BlockSpec((1, 64), lambda i: (0, 0)),
            pl.BlockSpec((64, 2), lambda i: (0, 0)),
            pl.BlockSpec((1, 2), lambda i: (0, 0)),
            pl.BlockSpec((1, 2), lambda i: (0, 0)),
        ],
        out_specs=pl.BlockSpec((G, 2), lambda i: (0, 0)),
        out_shape=jax.ShapeDtypeStruct((G, 2), jnp.float32),
        scratch_shapes=[pltpu.VMEM((G, 3), jnp.float32)],
    )(x, ids3, W1, b1_2d, Wc, bc, bc0)


# speedup vs baseline: 8.1735x; 1.1114x over previous
import jax
import jax.numpy as jnp
from jax.experimental import pallas as pl
from jax.experimental.pallas import tpu as pltpu

N = 100000
D = 128
G = 512
BLK = 2000
NBLK = N // BLK
WIN = 32


def _body(x_ref, ids_ref, w1_ref, b1_ref, wc_ref, bc_ref, bc0_ref, out_ref,
          acc_ref):
    i = pl.program_id(0)

    @pl.when(i == 0)
    def _init():
        acc_ref[...] = jnp.zeros_like(acc_ref)

    h = jnp.dot(x_ref[...], w1_ref[...], preferred_element_type=jnp.float32)
    h = jnp.maximum(h + b1_ref[...], 0.0)
    y = jnp.dot(h, wc_ref[...], preferred_element_type=jnp.float32)
    y3 = jnp.concatenate([y, jnp.ones((BLK, 1), jnp.float32)], axis=1)
    ids = ids_ref[0, :, :]

    b0 = ids_ref[0, 0, 0] // WIN
    b1 = ids_ref[0, 0, BLK - 1] // WIN
    iota_w = jax.lax.broadcasted_iota(jnp.int32, (WIN, BLK), 0)

    def _win(w, _):
        start = (b0 + w) * WIN
        onehot = (iota_w == ids - start).astype(jnp.float32)
        acc_ref[pl.ds(start, WIN), :] += jnp.dot(
            onehot, y3, preferred_element_type=jnp.float32)
        return 0

    jax.lax.fori_loop(0, b1 - b0 + 1, _win, 0)

    @pl.when(i == NBLK - 1)
    def _fin():
        acc = acc_ref[...]
        counts = acc[:, 2:3]
        z = acc[:, 0:2] / jnp.maximum(counts, 1.0) + bc_ref[...]
        z = jnp.where(counts > 0.0, z, bc0_ref[...])
        zmax = jnp.max(z, axis=0, keepdims=True)
        e = jnp.exp(z - zmax)
        out_ref[...] = e / jnp.sum(e, axis=0, keepdims=True)


def kernel(x, batch, W1, b1, W2, b2, Wl1, bl1, Wl2, bl2):
    ids3 = batch.astype(jnp.int32).reshape(NBLK, 1, BLK)
    Wm = Wl1 @ Wl2
    Wc = W2 @ Wm
    bc0 = bl1 @ Wl2 + bl2
    bc = (b2 @ Wm + bc0).reshape(1, 2)
    bc0 = bc0.reshape(1, 2)
    b1_2d = b1.reshape(1, 64)
    return pl.pallas_call(
        _body,
        grid=(NBLK,),
        in_specs=[
            pl.BlockSpec((BLK, D), lambda i: (i, 0)),
            pl.BlockSpec((1, 1, BLK), lambda i: (i, 0, 0)),
            pl.BlockSpec((D, 64), lambda i: (0, 0)),
            pl.BlockSpec((1, 64), lambda i: (0, 0)),
            pl.BlockSpec((64, 2), lambda i: (0, 0)),
            pl.BlockSpec((1, 2), lambda i: (0, 0)),
            pl.BlockSpec((1, 2), lambda i: (0, 0)),
        ],
        out_specs=pl.BlockSpec((G, 2), lambda i: (0, 0)),
        out_shape=jax.ShapeDtypeStruct((G, 2), jnp.float32),
        scratch_shapes=[pltpu.VMEM((G, 3), jnp.float32)],
    )(x, ids3, W1, b1_2d, Wc, bc, bc0)


# speedup vs baseline: 10.8385x; 1.3260x over previous
import jax
import jax.numpy as jnp
from jax.experimental import pallas as pl
from jax.experimental.pallas import tpu as pltpu

N = 100000
D = 128
G = 512
BLK = 4000
NBLK = N // BLK
WIN = 32


def _body(x_ref, ids_ref, w1_ref, b1_ref, wc_ref, bc_ref, bc0_ref, out_ref,
          acc_ref):
    i = pl.program_id(0)

    @pl.when(i == 0)
    def _init():
        acc_ref[...] = jnp.zeros_like(acc_ref)

    h = jnp.dot(x_ref[...].astype(jnp.bfloat16), w1_ref[...],
                preferred_element_type=jnp.float32)
    h = jnp.maximum(h + b1_ref[...], 0.0).astype(jnp.bfloat16)
    y = jnp.dot(h, wc_ref[...], preferred_element_type=jnp.float32)
    y3 = jnp.concatenate([y, jnp.ones((BLK, 1), jnp.float32)],
                         axis=1).astype(jnp.bfloat16)
    ids = ids_ref[0, :, :]

    b0 = ids_ref[0, 0, 0] // WIN
    b1 = ids_ref[0, 0, BLK - 1] // WIN
    iota_w = jax.lax.broadcasted_iota(jnp.int32, (WIN, BLK), 0)

    def _win(w, _):
        start = (b0 + w) * WIN
        onehot = (iota_w == ids - start).astype(jnp.bfloat16)
        acc_ref[pl.ds(start, WIN), :] += jnp.dot(
            onehot, y3, preferred_element_type=jnp.float32)
        return 0

    jax.lax.fori_loop(0, b1 - b0 + 1, _win, 0)

    @pl.when(i == NBLK - 1)
    def _fin():
        acc = acc_ref[...]
        counts = acc[:, 2:3]
        z = acc[:, 0:2] / jnp.maximum(counts, 1.0) + bc_ref[...]
        z = jnp.where(counts > 0.0, z, bc0_ref[...])
        zmax = jnp.max(z, axis=0, keepdims=True)
        e = jnp.exp(z - zmax)
        out_ref[...] = e / jnp.sum(e, axis=0, keepdims=True)


def kernel(x, batch, W1, b1, W2, b2, Wl1, bl1, Wl2, bl2):
    ids3 = batch.astype(jnp.int32).reshape(NBLK, 1, BLK)
    Wm = Wl1 @ Wl2
    Wc = (W2 @ Wm).astype(jnp.bfloat16)
    bc0 = bl1 @ Wl2 + bl2
    bc = (b2 @ Wm + bc0).reshape(1, 2)
    bc0 = bc0.reshape(1, 2)
    b1_2d = b1.reshape(1, 64)
    W1 = W1.astype(jnp.bfloat16)
    return pl.pallas_call(
        _body,
        grid=(NBLK,),
        in_specs=[
            pl.BlockSpec((BLK, D), lambda i: (i, 0)),
            pl.BlockSpec((1, 1, BLK), lambda i: (i, 0, 0)),
            pl.BlockSpec((D, 64), lambda i: (0, 0)),
            pl.BlockSpec((1, 64), lambda i: (0, 0)),
            pl.BlockSpec((64, 2), lambda i: (0, 0)),
            pl.BlockSpec((1, 2), lambda i: (0, 0)),
            pl.BlockSpec((1, 2), lambda i: (0, 0)),
        ],
        out_specs=pl.BlockSpec((G, 2), lambda i: (0, 0)),
        out_shape=jax.ShapeDtypeStruct((G, 2), jnp.float32),
        scratch_shapes=[pltpu.VMEM((G, 3), jnp.float32)],
    )(x, ids3, W1, b1_2d, Wc, bc, bc0)
